# baseline (speedup 1.0000x reference)
import functools

import jax
import jax.numpy as jnp
from jax import lax
from jax.experimental import pallas as pl
from jax.experimental.pallas import tpu as pltpu
from jax.experimental.pallas import tpu_sc as plsc

_DIM = 32
_NC, _NS = 2, 16
_NW = _NC * _NS
_SB = 8
_BB = 128

_mesh = plsc.VectorSubcoreMesh(
    core_axis_name="c", subcore_axis_name="s",
    num_cores=_NC, num_subcores=_NS)


_G = 7
_PER_W = 244
_GROUPS = [_G] * (_PER_W // _G) + ([_PER_W % _G] if _PER_W % _G else [])


def _transpose_group(slab_v, rows_v, slab_row0, rows_off0, n_slabs):
  iota32 = lax.iota(jnp.int32, 16) * _DIM

  @plsc.parallel_loop(0, n_slabs * _DIM * 8, unroll=16)
  def _(it):
    r = lax.shift_right_logical(it, 3)
    cg = lax.bitwise_and(it, 7)
    jj = lax.shift_right_logical(r, 5)
    d = lax.bitwise_and(r, 31)
    v = slab_v[slab_row0 + r, pl.ds(cg * 16, 16)]
    base = rows_off0 + (jj * _BB + cg * 16) * _DIM + d
    plsc.store_scatter(rows_v, [iota32 + base], v)


@jax.jit
def _linearize_table(table_t, tail_flat):
  v_total = table_t.shape[1]
  n_full = v_total // _BB
  n_tail = v_total - n_full * _BB

  @functools.partial(
      pl.kernel,
      out_type=jax.ShapeDtypeStruct((v_total * _DIM,), jnp.float32),
      mesh=_mesh,
      scratch_types=[
          pltpu.VMEM((2 * _G * _DIM, _BB), jnp.float32),
          pltpu.VMEM((2 * _G * _BB * _DIM,), jnp.float32),
          pltpu.SemaphoreType.DMA((2,)),
          pltpu.SemaphoreType.DMA((2,)),
      ],
      compiler_params=pltpu.CompilerParams(use_tc_tiling_on_sc=True,
                                           needs_layout_passes=False),
  )
  def linearize_kernel(tab_hbm, tail_hbm, out_hbm, slab_v, rows_v, rsem, ssem):
    wid = lax.axis_index("s") * _NC + lax.axis_index("c")
    base_j = wid * _PER_W
    starts = [sum(_GROUPS[:g]) for g in range(len(_GROUPS))]

    def issue_reads(g, b):
      c0 = (base_j + starts[g]) * _BB
      return [
          pltpu.async_copy(
              tab_hbm.at[:, pl.ds(c0 + jj * _BB, _BB)],
              slab_v.at[pl.ds((b * _G + jj) * _DIM, _DIM), :], rsem.at[b])
          for jj in range(_GROUPS[g])
      ]

    def issue_store(g, b):
      o0 = (base_j + starts[g]) * _BB * _DIM
      n = _GROUPS[g] * _BB * _DIM
      return pltpu.async_copy(rows_v.at[pl.ds(b * _G * _BB * _DIM, n)],
                              out_hbm.at[pl.ds(o0, n)], ssem.at[b])

    n_groups = len(_GROUPS)
    reads = {0: issue_reads(0, 0)}
    stores = {}
    for g in range(n_groups):
      b = g % 2
      for h in reads.pop(g):
        h.wait()
      if g + 1 < n_groups:
        reads[g + 1] = issue_reads(g + 1, 1 - b)
      if g >= 2:
        stores.pop(g - 2).wait()
      _transpose_group(slab_v, rows_v, b * _G * _DIM, b * _G * _BB * _DIM,
                       _GROUPS[g])
      stores[g] = issue_store(g, b)
    for g in sorted(stores):
      stores.pop(g).wait()

    @pl.when(wid < n_full - _NW * _PER_W)
    def _():
      j = _NW * _PER_W + wid
      pltpu.sync_copy(tab_hbm.at[:, pl.ds(j * _BB, _BB)],
                      slab_v.at[pl.ds(0, _DIM), :])
      _transpose_group(slab_v, rows_v, 0, 0, 1)
      pltpu.sync_copy(rows_v.at[pl.ds(0, _BB * _DIM)],
                      out_hbm.at[pl.ds(j * _BB * _DIM, _BB * _DIM)])

    @pl.when(wid == n_full - _NW * _PER_W)
    def _():
      pltpu.sync_copy(tail_hbm, rows_v.at[pl.ds(0, n_tail * _DIM)])
      pltpu.sync_copy(rows_v.at[pl.ds(0, n_tail * _DIM)],
                      out_hbm.at[pl.ds(n_full * _BB * _DIM, n_tail * _DIM)])

  return linearize_kernel(table_t, tail_flat)


@functools.partial(jax.jit, static_argnums=(2, 3))
def _gather_rows(idx_t, table_lin, s_total, b_total):
  n_units = s_total // _SB

  @functools.partial(
      pl.kernel,
      out_type=jax.ShapeDtypeStruct((s_total, b_total, _DIM), jnp.float32),
      mesh=_mesh,
      scratch_types=[
          pltpu.VMEM((2, _SB, _BB), jnp.int32),
          pltpu.VMEM((2, _SB, _BB, _DIM), jnp.float32),
          pltpu.SemaphoreType.DMA((2,)),
          pltpu.SemaphoreType.DMA((2,)),
      ],
      compiler_params=pltpu.CompilerParams(use_tc_tiling_on_sc=False),
  )
  def gather_kernel(idx_hbm, table_hbm, out_hbm, idx_v, rows_v, gsem, ssem):
    wid = lax.axis_index("s") * _NC + lax.axis_index("c")
    b0 = wid * _BB

    def issue_gather(k, b):
      pltpu.sync_copy(idx_hbm.at[pl.ds(k * _SB, _SB), pl.ds(b0, _BB)],
                      idx_v.at[b])
      return [
          pltpu.async_copy(table_hbm.at[idx_v.at[b, si]], rows_v.at[b, si],
                           gsem.at[b])
          for si in range(_SB)
      ]

    def issue_store(k, b):
      return pltpu.async_copy(
          rows_v.at[b],
          out_hbm.at[pl.ds(k * _SB, _SB), pl.ds(b0, _BB), :],
          ssem.at[b])

    gathers = {0: issue_gather(0, 0)}
    stores = {}
    for k in range(n_units):
      b = k % 2
      if k + 1 < n_units:
        if k >= 1:
          stores.pop(k - 1).wait()
        gathers[k + 1] = issue_gather(k + 1, 1 - b)
      for h in gathers.pop(k):
        h.wait()
      stores[k] = issue_store(k, b)
    for k in sorted(stores):
      stores.pop(k).wait()

  return gather_kernel(idx_t, table_lin)


def kernel(i, table):
  b_total, s_total = i.shape
  n_full = (table.shape[0] // _BB) * _BB
  tail_flat = table[n_full:].reshape(-1)
  table_lin = _linearize_table(table.T, tail_flat).reshape(table.shape)
  out_t = _gather_rows(i.T, table_lin, s_total, b_total)
  return out_t.transpose(1, 0, 2)


# speedup vs baseline: 1.7593x; 1.7593x over previous
import functools

import jax
import jax.numpy as jnp
from jax import lax
from jax.experimental import pallas as pl
from jax.experimental.pallas import tpu as pltpu
from jax.experimental.pallas import tpu_sc as plsc

_DIM = 32
_NC, _NS = 2, 16
_NW = _NC * _NS
_SB = 8
_BB = 128

_mesh = plsc.VectorSubcoreMesh(
    core_axis_name="c", subcore_axis_name="s",
    num_cores=_NC, num_subcores=_NS)


_G = 7
_PER_W = 244
_NG = _PER_W // _G


def _transpose_group(slab_v, rows_v, slab_row0, rows_off0, n_slabs):
  iota = lax.iota(jnp.int32, 16)

  @plsc.parallel_loop(0, n_slabs * 16, unroll=1)
  def _(blk):
    jj = lax.shift_right_logical(blk, 4)
    rem = lax.bitwise_and(blk, 15)
    db = lax.bitwise_and(rem, 1)
    cb16 = lax.shift_right_logical(rem, 1) * 16
    r_vec = iota + (slab_row0 + jj * _DIM + db * 16)
    sbase = rows_off0 + (jj * _BB + cb16) * _DIM + db * 16
    s_vec = iota + sbase
    for k in range(16):
      rot = lax.bitwise_and(iota + k, 15)
      v = plsc.load_gather(slab_v, [r_vec, rot + cb16])
      plsc.store_scatter(rows_v, [lax.shift_left(rot, 5) + s_vec], v)


@jax.jit
def _linearize_table(table_t, tail_flat):
  v_total = table_t.shape[1]
  n_full = v_total // _BB
  n_tail = v_total - n_full * _BB

  @functools.partial(
      pl.kernel,
      out_type=jax.ShapeDtypeStruct((v_total * _DIM,), jnp.float32),
      mesh=_mesh,
      scratch_types=[
          pltpu.VMEM((2 * _G * _DIM, _BB), jnp.float32),
          pltpu.VMEM((2 * _G * _BB * _DIM,), jnp.float32),
          pltpu.SemaphoreType.DMA((2,)),
          pltpu.SemaphoreType.DMA((2,)),
      ],
      compiler_params=pltpu.CompilerParams(use_tc_tiling_on_sc=True,
                                           needs_layout_passes=False),
  )
  def linearize_kernel(tab_hbm, tail_hbm, out_hbm, slab_v, rows_v, rsem, ssem):
    wid = lax.axis_index("s") * _NC + lax.axis_index("c")
    base_j = wid * _PER_W
    n_grp = _G * _BB * _DIM

    def issue_reads(g, b):
      c0 = (base_j + g * _G) * _BB
      for jj in range(_G):
        pltpu.async_copy(
            tab_hbm.at[:, pl.ds(c0 + jj * _BB, _BB)],
            slab_v.at[pl.ds((b * _G + jj) * _DIM, _DIM), :], rsem.at[b])

    def wait_reads(b):
      for jj in range(_G):
        pltpu.make_async_copy(
            tab_hbm.at[:, pl.ds(0, _BB)],
            slab_v.at[pl.ds((b * _G + jj) * _DIM, _DIM), :],
            rsem.at[b]).wait()

    def issue_store(g, b):
      o0 = (base_j + g * _G) * _BB * _DIM
      pltpu.async_copy(rows_v.at[pl.ds(b * n_grp, n_grp)],
                       out_hbm.at[pl.ds(o0, n_grp)], ssem.at[b])

    def wait_store(b):
      pltpu.make_async_copy(rows_v.at[pl.ds(b * n_grp, n_grp)],
                            out_hbm.at[pl.ds(0, n_grp)], ssem.at[b]).wait()

    def half(t, b, g):
      wait_reads(b)

      @pl.when(g + 1 < _NG)
      def _():
        issue_reads(g + 1, 1 - b)

      @pl.when(t >= 1)
      def _():
        wait_store(b)

      _transpose_group(slab_v, rows_v, b * _G * _DIM, b * n_grp, _G)
      issue_store(g, b)

    issue_reads(0, 0)

    def pair(t, carry):
      half(t, 0, 2 * t)
      half(t, 1, 2 * t + 1)
      return carry

    lax.fori_loop(0, _NG // 2, pair, 0)
    wait_store(0)
    wait_store(1)

    rem = _PER_W - _NG * _G
    c0 = (base_j + _NG * _G) * _BB
    for jj in range(rem):
      pltpu.async_copy(tab_hbm.at[:, pl.ds(c0 + jj * _BB, _BB)],
                       slab_v.at[pl.ds(jj * _DIM, _DIM), :], rsem.at[0])
    for jj in range(rem):
      pltpu.make_async_copy(tab_hbm.at[:, pl.ds(0, _BB)],
                            slab_v.at[pl.ds(jj * _DIM, _DIM), :],
                            rsem.at[0]).wait()
    _transpose_group(slab_v, rows_v, 0, 0, rem)
    pltpu.sync_copy(rows_v.at[pl.ds(0, rem * _BB * _DIM)],
                    out_hbm.at[pl.ds(c0 * _DIM, rem * _BB * _DIM)])

    @pl.when(wid < n_full - _NW * _PER_W)
    def _():
      j = _NW * _PER_W + wid
      pltpu.sync_copy(tab_hbm.at[:, pl.ds(j * _BB, _BB)],
                      slab_v.at[pl.ds(0, _DIM), :])
      _transpose_group(slab_v, rows_v, 0, 0, 1)
      pltpu.sync_copy(rows_v.at[pl.ds(0, _BB * _DIM)],
                      out_hbm.at[pl.ds(j * _BB * _DIM, _BB * _DIM)])

    @pl.when(wid == n_full - _NW * _PER_W)
    def _():
      pltpu.sync_copy(tail_hbm, rows_v.at[pl.ds(0, n_tail * _DIM)])
      pltpu.sync_copy(rows_v.at[pl.ds(0, n_tail * _DIM)],
                      out_hbm.at[pl.ds(n_full * _BB * _DIM, n_tail * _DIM)])

  return linearize_kernel(table_t, tail_flat)


@functools.partial(jax.jit, static_argnums=(2, 3))
def _gather_rows(idx_t, table_lin, s_total, b_total):
  n_units = s_total // _SB

  @functools.partial(
      pl.kernel,
      out_type=jax.ShapeDtypeStruct((s_total, b_total, _DIM), jnp.float32),
      mesh=_mesh,
      scratch_types=[
          pltpu.VMEM((2, _SB, _BB), jnp.int32),
          pltpu.VMEM((2, _SB, _BB, _DIM), jnp.float32),
          pltpu.SemaphoreType.DMA((2,)),
          pltpu.SemaphoreType.DMA((2,)),
      ],
      compiler_params=pltpu.CompilerParams(use_tc_tiling_on_sc=False),
  )
  def gather_kernel(idx_hbm, table_hbm, out_hbm, idx_v, rows_v, gsem, ssem):
    wid = lax.axis_index("s") * _NC + lax.axis_index("c")
    b0 = wid * _BB

    def issue_gather(k, b):
      pltpu.sync_copy(idx_hbm.at[pl.ds(k * _SB, _SB), pl.ds(b0, _BB)],
                      idx_v.at[b])
      return [
          pltpu.async_copy(table_hbm.at[idx_v.at[b, si]], rows_v.at[b, si],
                           gsem.at[b])
          for si in range(_SB)
      ]

    def issue_store(k, b):
      return pltpu.async_copy(
          rows_v.at[b],
          out_hbm.at[pl.ds(k * _SB, _SB), pl.ds(b0, _BB), :],
          ssem.at[b])

    gathers = {0: issue_gather(0, 0)}
    stores = {}
    for k in range(n_units):
      b = k % 2
      if k + 1 < n_units:
        if k >= 1:
          stores.pop(k - 1).wait()
        gathers[k + 1] = issue_gather(k + 1, 1 - b)
      for h in gathers.pop(k):
        h.wait()
      stores[k] = issue_store(k, b)
    for k in sorted(stores):
      stores.pop(k).wait()

  return gather_kernel(idx_t, table_lin)


def kernel(i, table):
  b_total, s_total = i.shape
  n_full = (table.shape[0] // _BB) * _BB
  tail_flat = table[n_full:].reshape(-1)
  table_lin = _linearize_table(table.T, tail_flat).reshape(table.shape)
  out_t = _gather_rows(i.T, table_lin, s_total, b_total)
  return out_t.transpose(1, 0, 2)


# speedup vs baseline: 3.2820x; 1.8656x over previous
import functools

import jax
import jax.numpy as jnp
from jax import lax
from jax.experimental import pallas as pl
from jax.experimental.pallas import tpu as pltpu
from jax.experimental.pallas import tpu_sc as plsc

_DIM = 32
_NC, _NS = 2, 16
_NW = _NC * _NS
_SB = 8
_BB = 128

_mesh = plsc.VectorSubcoreMesh(
    core_axis_name="c", subcore_axis_name="s",
    num_cores=_NC, num_subcores=_NS)


_G = 7
_PER_W = 244
_NG = _PER_W // _G


def _transpose_group(slab_v, rows_v, slab_row0, rows_off0, n_slabs):
  iota = lax.iota(jnp.int32, 16)

  @plsc.parallel_loop(0, n_slabs * 16, unroll=1)
  def _(blk):
    jj = lax.shift_right_logical(blk, 4)
    rem = lax.bitwise_and(blk, 15)
    db = lax.bitwise_and(rem, 1)
    cb16 = lax.shift_right_logical(rem, 1) * 16
    r_vec = iota + (slab_row0 + jj * _DIM + db * 16)
    sbase = rows_off0 + (jj * _BB + cb16) * _DIM + db * 16
    s_vec = iota + sbase
    for k in range(16):
      rot = lax.bitwise_and(iota + k, 15)
      v = plsc.load_gather(slab_v, [r_vec, rot + cb16])
      plsc.store_scatter(rows_v, [lax.shift_left(rot, 5) + s_vec], v)


@jax.jit
def _linearize_table(table_t, tail_flat):
  v_total = table_t.shape[1]
  n_full = v_total // _BB
  n_tail = v_total - n_full * _BB

  @functools.partial(
      pl.kernel,
      out_type=jax.ShapeDtypeStruct((v_total * _DIM,), jnp.float32),
      mesh=_mesh,
      scratch_types=[
          pltpu.VMEM((2 * _G * _DIM, _BB), jnp.float32),
          pltpu.VMEM((2 * _G * _BB * _DIM,), jnp.float32),
          pltpu.SemaphoreType.DMA((2,)),
          pltpu.SemaphoreType.DMA((2,)),
      ],
      compiler_params=pltpu.CompilerParams(use_tc_tiling_on_sc=True,
                                           needs_layout_passes=False),
  )
  def linearize_kernel(tab_hbm, tail_hbm, out_hbm, slab_v, rows_v, rsem, ssem):
    wid = lax.axis_index("s") * _NC + lax.axis_index("c")
    base_j = wid * _PER_W
    n_grp = _G * _BB * _DIM

    def issue_reads(g, b):
      c0 = (base_j + g * _G) * _BB
      for jj in range(_G):
        pltpu.async_copy(
            tab_hbm.at[:, pl.ds(c0 + jj * _BB, _BB)],
            slab_v.at[pl.ds((b * _G + jj) * _DIM, _DIM), :], rsem.at[b])

    def wait_reads(b):
      for jj in range(_G):
        pltpu.make_async_copy(
            tab_hbm.at[:, pl.ds(0, _BB)],
            slab_v.at[pl.ds((b * _G + jj) * _DIM, _DIM), :],
            rsem.at[b]).wait()

    def issue_store(g, b):
      o0 = (base_j + g * _G) * _BB * _DIM
      pltpu.async_copy(rows_v.at[pl.ds(b * n_grp, n_grp)],
                       out_hbm.at[pl.ds(o0, n_grp)], ssem.at[b])

    def wait_store(b):
      pltpu.make_async_copy(rows_v.at[pl.ds(b * n_grp, n_grp)],
                            out_hbm.at[pl.ds(0, n_grp)], ssem.at[b]).wait()

    def half(t, b, g):
      wait_reads(b)

      @pl.when(g + 1 < _NG)
      def _():
        issue_reads(g + 1, 1 - b)

      @pl.when(t >= 1)
      def _():
        wait_store(b)

      _transpose_group(slab_v, rows_v, b * _G * _DIM, b * n_grp, _G)
      issue_store(g, b)

    issue_reads(0, 0)

    def pair(t, carry):
      half(t, 0, 2 * t)
      half(t, 1, 2 * t + 1)
      return carry

    lax.fori_loop(0, _NG // 2, pair, 0)
    wait_store(0)
    wait_store(1)

    rem = _PER_W - _NG * _G
    c0 = (base_j + _NG * _G) * _BB
    for jj in range(rem):
      pltpu.async_copy(tab_hbm.at[:, pl.ds(c0 + jj * _BB, _BB)],
                       slab_v.at[pl.ds(jj * _DIM, _DIM), :], rsem.at[0])
    for jj in range(rem):
      pltpu.make_async_copy(tab_hbm.at[:, pl.ds(0, _BB)],
                            slab_v.at[pl.ds(jj * _DIM, _DIM), :],
                            rsem.at[0]).wait()
    _transpose_group(slab_v, rows_v, 0, 0, rem)
    pltpu.sync_copy(rows_v.at[pl.ds(0, rem * _BB * _DIM)],
                    out_hbm.at[pl.ds(c0 * _DIM, rem * _BB * _DIM)])

    @pl.when(wid < n_full - _NW * _PER_W)
    def _():
      j = _NW * _PER_W + wid
      pltpu.sync_copy(tab_hbm.at[:, pl.ds(j * _BB, _BB)],
                      slab_v.at[pl.ds(0, _DIM), :])
      _transpose_group(slab_v, rows_v, 0, 0, 1)
      pltpu.sync_copy(rows_v.at[pl.ds(0, _BB * _DIM)],
                      out_hbm.at[pl.ds(j * _BB * _DIM, _BB * _DIM)])

    @pl.when(wid == n_full - _NW * _PER_W)
    def _():
      pltpu.sync_copy(tail_hbm, rows_v.at[pl.ds(0, n_tail * _DIM)])
      pltpu.sync_copy(rows_v.at[pl.ds(0, n_tail * _DIM)],
                      out_hbm.at[pl.ds(n_full * _BB * _DIM, n_tail * _DIM)])

  return linearize_kernel(table_t, tail_flat)


@functools.partial(jax.jit, static_argnums=(2, 3))
def _gather_rows(idx_t, table_lin, s_total, b_total):
  n_units = s_total // _SB
  n_bt = b_total // _BB

  @functools.partial(
      pl.kernel,
      out_type=jax.ShapeDtypeStruct((s_total, _DIM // 8, n_bt, 8, _BB),
                                    jnp.float32),
      mesh=_mesh,
      scratch_types=[
          pltpu.VMEM((2, _SB, _BB), jnp.int32),
          pltpu.VMEM((2, _SB, _BB, _DIM), jnp.float32),
          pltpu.VMEM((_SB, _DIM // 8, 8, _BB), jnp.float32),
          pltpu.SemaphoreType.DMA((2,)),
          pltpu.SemaphoreType.DMA,
      ],
      compiler_params=pltpu.CompilerParams(use_tc_tiling_on_sc=False,
                                           needs_layout_passes=False),
  )
  def gather_kernel(idx_hbm, table_hbm, out_hbm, idx_v, rows_v, slab_v,
                    gsem, ssem):
    wid = lax.axis_index("s") * _NC + lax.axis_index("c")
    b0 = wid * _BB
    iota = lax.iota(jnp.int32, 16)

    def issue_gather(k, b):
      pltpu.sync_copy(idx_hbm.at[pl.ds(k * _SB, _SB), pl.ds(b0, _BB)],
                      idx_v.at[b])
      return [
          pltpu.async_copy(table_hbm.at[idx_v.at[b, si]], rows_v.at[b, si],
                           gsem.at[b])
          for si in range(_SB)
      ]

    def transpose_unit(b):
      @plsc.parallel_loop(0, _SB * 2 * 8, unroll=1)
      def _(blk):
        s = lax.shift_right_logical(blk, 4)
        rem = lax.bitwise_and(blk, 15)
        d0 = lax.shift_right_logical(rem, 3) * 16
        bb0 = lax.bitwise_and(rem, 7) * 16
        s_vec = jnp.full((16,), s, jnp.int32)
        d_vec = iota + d0
        db_vec = lax.shift_right_logical(d_vec, 3)
        d8_vec = lax.bitwise_and(d_vec, 7)
        for k in range(16):
          rot = lax.bitwise_and(iota + k, 15)
          bb_vec = rot + bb0
          v = plsc.load_gather(rows_v.at[b], [s_vec, bb_vec, d_vec])
          plsc.store_scatter(slab_v, [s_vec, db_vec, d8_vec, bb_vec], v)

    def issue_store(k):
      pltpu.async_copy(slab_v, out_hbm.at[pl.ds(k * _SB, _SB), :,
                                          wid, :, :], ssem)

    def wait_gathers(b):
      for si in range(_SB):
        pltpu.make_async_copy(table_hbm.at[idx_v.at[b, si]],
                              rows_v.at[b, si], gsem.at[b]).wait()

    def wait_store():
      pltpu.make_async_copy(slab_v, out_hbm.at[pl.ds(0, _SB), :, wid, :, :],
                            ssem).wait()

    issue_gather(0, 0)

    def pair(t, carry):
      k0 = 2 * t
      issue_gather(k0 + 1, 1)
      wait_gathers(0)

      @pl.when(t >= 1)
      def _():
        wait_store()

      transpose_unit(0)
      issue_store(k0)

      issue_gather(k0 + 2, 0)
      wait_gathers(1)
      wait_store()
      transpose_unit(1)
      issue_store(k0 + 1)
      return carry

    lax.fori_loop(0, n_units // 2, pair, 0)

    wait_gathers(0)
    wait_store()
    transpose_unit(0)
    issue_store(n_units - 1)
    wait_store()

  return gather_kernel(idx_t, table_lin)


def kernel(i, table):
  b_total, s_total = i.shape
  n_full = (table.shape[0] // _BB) * _BB
  tail_flat = table[n_full:].reshape(-1)
  table_lin = _linearize_table(table.T, tail_flat).reshape(table.shape)
  out_p = _gather_rows(i.T, table_lin, s_total, b_total)
  return out_p.transpose(2, 4, 0, 1, 3).reshape(b_total, s_total, _DIM)


# speedup vs baseline: 4.3370x; 1.3214x over previous
import functools

import jax
import jax.numpy as jnp
from jax import lax
from jax.experimental import pallas as pl
from jax.experimental.pallas import tpu as pltpu
from jax.experimental.pallas import tpu_sc as plsc

_DIM = 32
_NC, _NS = 2, 16
_NW = _NC * _NS
_SB = 8
_BB = 128

_mesh = plsc.VectorSubcoreMesh(
    core_axis_name="c", subcore_axis_name="s",
    num_cores=_NC, num_subcores=_NS)


_G = 7
_PER_W = 244
_NG = _PER_W // _G


def _transpose_group(slab_v, rows_v, slab_row0, rows_off0, n_slabs):
  iota = lax.iota(jnp.int32, 16)

  @plsc.parallel_loop(0, n_slabs * 16, unroll=2)
  def _(blk):
    jj = lax.shift_right_logical(blk, 4)
    rem = lax.bitwise_and(blk, 15)
    db = lax.bitwise_and(rem, 1)
    cb16 = lax.shift_right_logical(rem, 1) * 16
    r_vec = iota + (slab_row0 + jj * _DIM + db * 16)
    sbase = rows_off0 + (jj * _BB + cb16) * _DIM + db * 16
    s_vec = iota + sbase
    for k in range(16):
      rot = lax.bitwise_and(iota + k, 15)
      v = plsc.load_gather(slab_v, [r_vec, rot + cb16])
      plsc.store_scatter(rows_v, [lax.shift_left(rot, 5) + s_vec], v)


@jax.jit
def _linearize_table(table_t, tail_flat):
  v_total = table_t.shape[1]
  n_full = v_total // _BB
  n_tail = v_total - n_full * _BB

  @functools.partial(
      pl.kernel,
      out_type=jax.ShapeDtypeStruct((v_total * _DIM,), jnp.float32),
      mesh=_mesh,
      scratch_types=[
          pltpu.VMEM((2 * _G * _DIM, _BB), jnp.float32),
          pltpu.VMEM((2 * _G * _BB * _DIM,), jnp.float32),
          pltpu.SemaphoreType.DMA((2,)),
          pltpu.SemaphoreType.DMA((2,)),
      ],
      compiler_params=pltpu.CompilerParams(use_tc_tiling_on_sc=True,
                                           needs_layout_passes=False),
  )
  def linearize_kernel(tab_hbm, tail_hbm, out_hbm, slab_v, rows_v, rsem, ssem):
    wid = lax.axis_index("s") * _NC + lax.axis_index("c")
    base_j = wid * _PER_W
    n_grp = _G * _BB * _DIM

    def issue_reads(g, b):
      c0 = (base_j + g * _G) * _BB
      for jj in range(_G):
        pltpu.async_copy(
            tab_hbm.at[:, pl.ds(c0 + jj * _BB, _BB)],
            slab_v.at[pl.ds((b * _G + jj) * _DIM, _DIM), :], rsem.at[b])

    def wait_reads(b):
      for jj in range(_G):
        pltpu.make_async_copy(
            tab_hbm.at[:, pl.ds(0, _BB)],
            slab_v.at[pl.ds((b * _G + jj) * _DIM, _DIM), :],
            rsem.at[b]).wait()

    def issue_store(g, b):
      o0 = (base_j + g * _G) * _BB * _DIM
      pltpu.async_copy(rows_v.at[pl.ds(b * n_grp, n_grp)],
                       out_hbm.at[pl.ds(o0, n_grp)], ssem.at[b])

    def wait_store(b):
      pltpu.make_async_copy(rows_v.at[pl.ds(b * n_grp, n_grp)],
                            out_hbm.at[pl.ds(0, n_grp)], ssem.at[b]).wait()

    def half(t, b, g):
      wait_reads(b)

      @pl.when(g + 1 < _NG)
      def _():
        issue_reads(g + 1, 1 - b)

      @pl.when(t >= 1)
      def _():
        wait_store(b)

      _transpose_group(slab_v, rows_v, b * _G * _DIM, b * n_grp, _G)
      issue_store(g, b)

    issue_reads(0, 0)

    def pair(t, carry):
      half(t, 0, 2 * t)
      half(t, 1, 2 * t + 1)
      return carry

    lax.fori_loop(0, _NG // 2, pair, 0)
    wait_store(0)
    wait_store(1)

    rem = _PER_W - _NG * _G
    c0 = (base_j + _NG * _G) * _BB
    for jj in range(rem):
      pltpu.async_copy(tab_hbm.at[:, pl.ds(c0 + jj * _BB, _BB)],
                       slab_v.at[pl.ds(jj * _DIM, _DIM), :], rsem.at[0])
    for jj in range(rem):
      pltpu.make_async_copy(tab_hbm.at[:, pl.ds(0, _BB)],
                            slab_v.at[pl.ds(jj * _DIM, _DIM), :],
                            rsem.at[0]).wait()
    _transpose_group(slab_v, rows_v, 0, 0, rem)
    pltpu.sync_copy(rows_v.at[pl.ds(0, rem * _BB * _DIM)],
                    out_hbm.at[pl.ds(c0 * _DIM, rem * _BB * _DIM)])

    @pl.when(wid < n_full - _NW * _PER_W)
    def _():
      j = _NW * _PER_W + wid
      pltpu.sync_copy(tab_hbm.at[:, pl.ds(j * _BB, _BB)],
                      slab_v.at[pl.ds(0, _DIM), :])
      _transpose_group(slab_v, rows_v, 0, 0, 1)
      pltpu.sync_copy(rows_v.at[pl.ds(0, _BB * _DIM)],
                      out_hbm.at[pl.ds(j * _BB * _DIM, _BB * _DIM)])

    @pl.when(wid == n_full - _NW * _PER_W)
    def _():
      pltpu.sync_copy(tail_hbm, rows_v.at[pl.ds(0, n_tail * _DIM)])
      pltpu.sync_copy(rows_v.at[pl.ds(0, n_tail * _DIM)],
                      out_hbm.at[pl.ds(n_full * _BB * _DIM, n_tail * _DIM)])

  return linearize_kernel(table_t, tail_flat)


@functools.partial(jax.jit, static_argnums=(2, 3))
def _gather_rows(idx_t, table_lin, s_total, b_total):
  n_units = s_total // _SB
  n_bt = b_total // _BB

  @functools.partial(
      pl.kernel,
      out_type=jax.ShapeDtypeStruct((s_total, _DIM // 8, n_bt, 8, _BB),
                                    jnp.float32),
      mesh=_mesh,
      scratch_types=[
          pltpu.VMEM((2, _SB, _BB), jnp.int32),
          pltpu.VMEM((2, _SB, _BB, _DIM), jnp.float32),
          pltpu.VMEM((_SB, _DIM // 8, 8, _BB), jnp.float32),
          pltpu.SemaphoreType.DMA((2,)),
          pltpu.SemaphoreType.DMA,
      ],
      compiler_params=pltpu.CompilerParams(use_tc_tiling_on_sc=False,
                                           needs_layout_passes=False),
  )
  def gather_kernel(idx_hbm, table_hbm, out_hbm, idx_v, rows_v, slab_v,
                    gsem, ssem):
    wid = lax.axis_index("s") * _NC + lax.axis_index("c")
    b0 = wid * _BB
    iota = lax.iota(jnp.int32, 16)

    def issue_gather(k, b):
      pltpu.sync_copy(idx_hbm.at[pl.ds(k * _SB, _SB), pl.ds(b0, _BB)],
                      idx_v.at[b])
      return [
          pltpu.async_copy(table_hbm.at[idx_v.at[b, si]], rows_v.at[b, si],
                           gsem.at[b])
          for si in range(_SB)
      ]

    def transpose_unit(b):
      @plsc.parallel_loop(0, _SB * 2 * 8, unroll=2)
      def _(blk):
        s = lax.shift_right_logical(blk, 4)
        rem = lax.bitwise_and(blk, 15)
        d0 = lax.shift_right_logical(rem, 3) * 16
        bb0 = lax.bitwise_and(rem, 7) * 16
        s_vec = jnp.full((16,), s, jnp.int32)
        d_vec = iota + d0
        db_vec = lax.shift_right_logical(d_vec, 3)
        d8_vec = lax.bitwise_and(d_vec, 7)
        for k in range(16):
          rot = lax.bitwise_and(iota + k, 15)
          bb_vec = rot + bb0
          v = plsc.load_gather(rows_v.at[b], [s_vec, bb_vec, d_vec])
          plsc.store_scatter(slab_v, [s_vec, db_vec, d8_vec, bb_vec], v)

    def issue_store(k):
      pltpu.async_copy(slab_v, out_hbm.at[pl.ds(k * _SB, _SB), :,
                                          wid, :, :], ssem)

    def wait_gathers(b):
      for si in range(_SB):
        pltpu.make_async_copy(table_hbm.at[idx_v.at[b, si]],
                              rows_v.at[b, si], gsem.at[b]).wait()

    def wait_store():
      pltpu.make_async_copy(slab_v, out_hbm.at[pl.ds(0, _SB), :, wid, :, :],
                            ssem).wait()

    issue_gather(0, 0)

    def pair(t, carry):
      k0 = 2 * t
      issue_gather(k0 + 1, 1)
      wait_gathers(0)

      @pl.when(t >= 1)
      def _():
        wait_store()

      transpose_unit(0)
      issue_store(k0)

      issue_gather(k0 + 2, 0)
      wait_gathers(1)
      wait_store()
      transpose_unit(1)
      issue_store(k0 + 1)
      return carry

    lax.fori_loop(0, n_units // 2, pair, 0)

    wait_gathers(0)
    wait_store()
    transpose_unit(0)
    issue_store(n_units - 1)
    wait_store()

  return gather_kernel(idx_t, table_lin)


def kernel(i, table):
  b_total, s_total = i.shape
  n_full = (table.shape[0] // _BB) * _BB
  tail_flat = table[n_full:].reshape(-1)
  table_lin = _linearize_table(table.T, tail_flat).reshape(table.shape)
  out_p = _gather_rows(i.T, table_lin, s_total, b_total)
  return out_p.transpose(2, 4, 0, 1, 3).reshape(b_total, s_total, _DIM)
